# Initial kernel scaffold; baseline (speedup 1.0000x reference)
import jax
import jax.numpy as jnp
from jax.experimental import pallas as pl


def kernel(feat_0_index, feat_0_value, feat_1_index, feat_1_value, feat_2_index, feat_2_value, feat_3_index, feat_3_value, feat_4_index, feat_4_value, feat_5_index, feat_5_value, feat_6_index, feat_6_value, feat_7_index, feat_7_value, feat_8_index, feat_8_value, feat_9_index, feat_9_value, feat_10_index, feat_10_value, feat_11_index, feat_11_value, feat_12_index, feat_12_value, feat_13_index, feat_13_value, feat_14_index, feat_14_value, feat_15_index, feat_15_value, feat_16_index, feat_16_value, feat_17_index, feat_17_value, feat_18_index, feat_18_value, feat_19_index, feat_19_value, feat_20_index, feat_20_value, feat_21_index, feat_21_value, feat_22_index, feat_22_value, feat_23_index, feat_23_value, feat_24_index, feat_24_value, feat_25_index, feat_25_value, emb_tables, wide_tables):
    raise NotImplementedError("write your pallas kernel here")


# speedup vs baseline: 2.4219x; 2.4219x over previous
import functools

import jax
import jax.numpy as jnp
from jax import lax
from jax.experimental import pallas as pl
from jax.experimental.pallas import tpu as pltpu
from jax.experimental.pallas import tpu_sc as plsc

_F = 26
_V = 100000
_D = 16
_B = 4096
_T = 20

_NC = 2
_NS = 16
_NW = _NC * _NS
_G_TOT = _F * _B
_GPW = _G_TOT // _NW
_SG = 128
_STEPS = _GPW // _SG
_LPS = _SG * _T
_NJ = _LPS // 128
_N = _G_TOT * _T


def _sc_body(gidx_hbm, gidxt_hbm, vals_hbm, emb_hbm, wide_hbm, oemb_hbm,
             owide_hbm, idx_v, idxt_v, val_v, rows_v, w_v, oemb_v, owide_v,
             sem_e, sem_w):
    wid = lax.axis_index("s") * _NC + lax.axis_index("c")

    def step(s, carry):
        g0 = wid * _GPW + s * _SG
        i0 = g0 * _T

        pltpu.sync_copy(gidx_hbm.at[pl.ds(i0, _LPS)], idx_v)
        pltpu.sync_copy(gidxt_hbm.at[pl.ds(i0, _LPS)], idxt_v)
        pltpu.sync_copy(vals_hbm.at[pl.ds(i0, _LPS)], val_v)

        cps = []
        for j in range(_NJ):
            cps.append(pltpu.async_copy(
                emb_hbm.at[idx_v.at[pl.ds(j * 128, 128)]],
                rows_v.at[pl.ds(j * 128, 128)], sem_e))
            cps.append(pltpu.async_copy(
                wide_hbm.at[idxt_v.at[pl.ds(j * 128, 128)]],
                w_v.at[pl.ds(j * 128, 128)], sem_w))
        for c in cps:
            c.wait()

        def grp(g, c2):
            b = g * _T
            lo = val_v[pl.ds(b, 16)]
            hi = val_v[pl.ds(b + 4, 16)]
            acc = rows_v[b, :] * lo[0]
            for t in range(1, 16):
                acc = acc + rows_v[b + t, :] * lo[t]
            for t in range(16, _T):
                acc = acc + rows_v[b + t, :] * hi[t - 4]
            oemb_v[g, :] = acc
            return c2

        lax.fori_loop(0, _SG, grp, 0)

        def wgrp(gb, c2):
            o = gb * 16
            acc = w_v[pl.ds(o, 16)]
            for t in range(1, _T):
                acc = acc + w_v[pl.ds(t * 128 + o, 16)]
            owide_v[pl.ds(o, 16)] = acc
            return c2

        lax.fori_loop(0, _SG // 16, wgrp, 0)

        pltpu.sync_copy(oemb_v, oemb_hbm.at[pl.ds(g0, _SG)])
        pltpu.sync_copy(owide_v, owide_hbm.at[pl.ds(g0, _SG)])
        return carry

    lax.fori_loop(0, _STEPS, step, 0)


_sc_pool = functools.partial(
    pl.kernel,
    out_type=[jax.ShapeDtypeStruct((_G_TOT, _D), jnp.float32),
              jax.ShapeDtypeStruct((_G_TOT,), jnp.float32)],
    mesh=plsc.VectorSubcoreMesh(core_axis_name="c", subcore_axis_name="s"),
    compiler_params=pltpu.CompilerParams(use_tc_tiling_on_sc=False),
    scratch_types=[
        pltpu.VMEM((_LPS,), jnp.int32),
        pltpu.VMEM((_LPS,), jnp.int32),
        pltpu.VMEM((_LPS,), jnp.float32),
        pltpu.VMEM((_LPS, _D), jnp.float32),
        pltpu.VMEM((_LPS,), jnp.float32),
        pltpu.VMEM((_SG, _D), jnp.float32),
        pltpu.VMEM((_SG,), jnp.float32),
        pltpu.SemaphoreType.DMA,
        pltpu.SemaphoreType.DMA,
    ],
)(_sc_body)


def kernel(feat_0_index, feat_0_value, feat_1_index, feat_1_value, feat_2_index, feat_2_value, feat_3_index, feat_3_value, feat_4_index, feat_4_value, feat_5_index, feat_5_value, feat_6_index, feat_6_value, feat_7_index, feat_7_value, feat_8_index, feat_8_value, feat_9_index, feat_9_value, feat_10_index, feat_10_value, feat_11_index, feat_11_value, feat_12_index, feat_12_value, feat_13_index, feat_13_value, feat_14_index, feat_14_value, feat_15_index, feat_15_value, feat_16_index, feat_16_value, feat_17_index, feat_17_value, feat_18_index, feat_18_value, feat_19_index, feat_19_value, feat_20_index, feat_20_value, feat_21_index, feat_21_value, feat_22_index, feat_22_value, feat_23_index, feat_23_value, feat_24_index, feat_24_value, feat_25_index, feat_25_value, emb_tables, wide_tables):
    feats = list(locals().values())
    idxs = [feats[2 * i] for i in range(_F)]
    vals = [feats[2 * i + 1] for i in range(_F)]

    idx = jnp.stack([a.reshape(_B * _T) for a in idxs])
    offs = (jnp.arange(_F, dtype=jnp.int32) * _V).reshape(_F, 1)
    gidx = (idx % _V + offs).reshape(_N)
    gidxt = gidx.reshape(_G_TOT // _SG, _SG, _T).transpose(0, 2, 1).reshape(_N)
    val = jnp.stack([a.reshape(_B * _T) for a in vals]).reshape(_N)

    oemb, owide = _sc_pool(gidx, gidxt, val,
                           emb_tables.reshape(_F * _V, _D),
                           wide_tables.reshape(_F * _V))

    emb_tensor = oemb.reshape(_F, _B, _D).transpose(1, 0, 2)
    wide_tensor = owide.reshape(_F, _B).transpose(1, 0)
    return (wide_tensor, emb_tensor)
